# baseline (speedup 1.0000x reference)
import functools

import jax
import jax.numpy as jnp
from jax import lax
from jax.experimental import pallas as pl
from jax.experimental.pallas import tpu as pltpu
from jax.experimental.pallas import tpu_sc as plsc

_TOPK = 5
_RADIUS = 5
_ROWS = 64
_N = 32768
_L = 16
_C = 256
_NCHUNK = _N // _C
_NVPC = _C // _L
_BIG = 1 << 30
_ZB = 4096

_mesh = plsc.VectorSubcoreMesh(
    core_axis_name="c", subcore_axis_name="s", num_cores=2, num_subcores=16
)


def _rescan_chunk(row_v, cmax_v, c):
    base = c * _C
    acc = row_v[pl.ds(base, _L)]
    for i in range(1, _NVPC):
        acc = jnp.maximum(acc, row_v[pl.ds(base + i * _L, _L)])
    cmax_v[pl.ds(c * _L, _L)] = acc


def _compute_picks(row_v, cmax_v, iota):
    def _cbody(c, gmin):
        base = c * _C
        acc = row_v[pl.ds(base, _L)]
        accmin = acc
        for i in range(1, _NVPC):
            v = row_v[pl.ds(base + i * _L, _L)]
            acc = jnp.maximum(acc, v)
            accmin = jnp.minimum(accmin, v)
        cmax_v[pl.ds(c * _L, _L)] = acc
        return jnp.minimum(gmin, accmin)

    gminv = lax.fori_loop(
        0, _NCHUNK, _cbody, jnp.full((_L,), jnp.inf, jnp.float32)
    )
    m = jnp.min(gminv)
    m_v = jnp.full((_L,), m, jnp.float32)

    picks = []
    for t in range(_TOPK):
        def _abody(i, acc):
            for u in range(8):
                acc = jnp.maximum(acc, cmax_v[pl.ds((i * 8 + u) * _L, _L)])
            return acc

        maxacc = lax.fori_loop(
            0, _NCHUNK // 8, _abody, jnp.full((_L,), -jnp.inf, jnp.float32)
        )
        big_m = jnp.max(maxacc)

        def _bbody(i, acc):
            for u in range(8):
                c = i * 8 + u
                cm = cmax_v[pl.ds(c * _L, _L)]
                acc = jnp.minimum(acc, jnp.where(cm == big_m, c, _BIG))
            return acc

        cidxv = lax.fori_loop(
            0, _NCHUNK // 8, _bbody, jnp.full((_L,), _BIG, jnp.int32)
        )
        cidx = jnp.min(cidxv)

        base = cidx * _C
        idxacc = jnp.full((_L,), _BIG, jnp.int32)
        for i in range(_NVPC):
            v = row_v[pl.ds(base + i * _L, _L)]
            idxacc = jnp.minimum(
                idxacc, jnp.where(v == big_m, base + i * _L + iota, _BIG)
            )
        gidx = jnp.min(idxacc)
        picks.append(gidx)

        widx = gidx - _RADIUS + iota
        wmask = (iota < 2 * _RADIUS) & (gidx >= _RADIUS) & (widx < _N)
        widx_c = jnp.clip(widx, 0, _N - 1)
        plsc.store_scatter(row_v, [widx_c], m_v, mask=wmask)

        if t < _TOPK - 1:
            ws = jnp.maximum(gidx - _RADIUS, 0)
            we = jnp.minimum(gidx + _RADIUS, _N) - 1
            _rescan_chunk(row_v, cmax_v, ws // _C)
            _rescan_chunk(row_v, cmax_v, we // _C)

    pv = jnp.full((_L,), picks[0], jnp.int32)
    for t in range(1, _TOPK):
        pv = jnp.where(iota == t, picks[t], pv)
    return pv


@functools.partial(
    pl.kernel,
    out_type=jax.ShapeDtypeStruct((_ROWS * _N,), jnp.float32),
    mesh=_mesh,
    compiler_params=pltpu.CompilerParams(needs_layout_passes=False),
    scratch_types=[
        pltpu.VMEM((_N,), jnp.float32),
        pltpu.VMEM((_N,), jnp.float32),
        pltpu.VMEM((_ZB,), jnp.float32),
        pltpu.VMEM((_NCHUNK * _L,), jnp.float32),
        pltpu.VMEM((_L,), jnp.int32),
        pltpu.VMEM((_L,), jnp.float32),
        pltpu.SemaphoreType.DMA,
        pltpu.SemaphoreType.DMA,
        pltpu.SemaphoreType.DMA,
        pltpu.SemaphoreType.DMA,
        pltpu.SemaphoreType.DMA,
    ],
)
def _wta_sc(x_hbm, out_hbm, row0_v, row1_v, zeros_v, cmax_v, picks_v,
            ones_v, sem_in0, sem_in1, sem_z0, sem_z1, sem_sc):
    wid = lax.axis_index("s") * 2 + lax.axis_index("c")
    ra = wid * 2
    rb = ra + 1
    iota = lax.iota(jnp.int32, _L)
    zero_v = jnp.zeros((_L,), jnp.float32)

    in0 = pltpu.async_copy(x_hbm.at[ra], row0_v, sem_in0)
    in1 = pltpu.async_copy(x_hbm.at[rb], row1_v, sem_in1)

    def _zbody(i, carry):
        for u in range(16):
            zeros_v[pl.ds(i * 256 + u * _L, _L)] = zero_v
        return carry

    lax.fori_loop(0, _ZB // 256, _zbody, jnp.int32(0))
    ones_v[...] = jnp.ones((_L,), jnp.float32)

    z0 = [
        pltpu.async_copy(
            zeros_v, out_hbm.at[pl.ds(ra * _N + j * _ZB, _ZB)], sem_z0
        )
        for j in range(_N // _ZB)
    ]
    z1 = [
        pltpu.async_copy(
            zeros_v, out_hbm.at[pl.ds(rb * _N + j * _ZB, _ZB)], sem_z1
        )
        for j in range(_N // _ZB)
    ]

    in0.wait()
    pv0 = _compute_picks(row0_v, cmax_v, iota)
    picks_v[...] = ra * _N + pv0
    for d in z0:
        d.wait()
    pltpu.async_copy(ones_v, out_hbm.at[picks_v], sem_sc).wait()

    in1.wait()
    pv1 = _compute_picks(row1_v, cmax_v, iota)
    picks_v[...] = rb * _N + pv1
    for d in z1:
        d.wait()
    pltpu.async_copy(ones_v, out_hbm.at[picks_v], sem_sc).wait()


def kernel(x):
    return _wta_sc(x).reshape(_ROWS, _N)


# speedup vs baseline: 1.4177x; 1.4177x over previous
import functools

import jax
import jax.numpy as jnp
from jax import lax
from jax.experimental import pallas as pl
from jax.experimental.pallas import tpu as pltpu
from jax.experimental.pallas import tpu_sc as plsc

_TOPK = 5
_RADIUS = 5
_ROWS = 64
_N = 32768
_L = 16
_C = 256
_NCHUNK = _N // _C
_NVPC = _C // _L
_NSUP = 8
_CPS = _NCHUNK // _NSUP
_SEC = _N // _NSUP
_BIG = 1 << 30

_mesh = plsc.VectorSubcoreMesh(
    core_axis_name="c", subcore_axis_name="s", num_cores=2, num_subcores=16
)


def _rescan_chunk(row_v, cmax_v, c):
    base = c * _C
    acc = row_v[pl.ds(base, _L)]
    for i in range(1, _NVPC):
        acc = jnp.maximum(acc, row_v[pl.ds(base + i * _L, _L)])
    cmax_v[pl.ds(c * _L, _L)] = acc


def _rebuild_super(cmax_v, smax_v, s):
    base = s * _CPS * _L
    acc = cmax_v[pl.ds(base, _L)]
    for k in range(1, _CPS):
        acc = jnp.maximum(acc, cmax_v[pl.ds(base + k * _L, _L)])
    smax_v[pl.ds(s * _L, _L)] = acc


def _compute_picks(row_v, cmax_v, smax_v, iota, sec_dmas):
    sec_dmas.wait()
    gminv = jnp.full((_L,), jnp.inf, jnp.float32)
    for s in range(_NSUP):

        def _cbody(i, carry):
            sacc, gmin = carry
            base = (s * _CPS + i) * _C
            acc = row_v[pl.ds(base, _L)]
            accmin = acc
            for k in range(1, _NVPC):
                v = row_v[pl.ds(base + k * _L, _L)]
                acc = jnp.maximum(acc, v)
                accmin = jnp.minimum(accmin, v)
            cmax_v[pl.ds((s * _CPS + i) * _L, _L)] = acc
            return jnp.maximum(sacc, acc), jnp.minimum(gmin, accmin)

        sacc, gminv = lax.fori_loop(
            0, _CPS, _cbody,
            (jnp.full((_L,), -jnp.inf, jnp.float32), gminv),
        )
        smax_v[pl.ds(s * _L, _L)] = sacc

    m = jnp.min(gminv)
    m_v = jnp.full((_L,), m, jnp.float32)

    picks = []
    for t in range(_TOPK):
        macc = smax_v[pl.ds(0, _L)]
        for s in range(1, _NSUP):
            macc = jnp.maximum(macc, smax_v[pl.ds(s * _L, _L)])
        big_m = jnp.max(macc)

        sidxv = jnp.full((_L,), _BIG, jnp.int32)
        for s in range(_NSUP):
            sm = smax_v[pl.ds(s * _L, _L)]
            sidxv = jnp.minimum(sidxv, jnp.where(sm == big_m, s, _BIG))
        sidx = jnp.min(sidxv)

        cbase = sidx * _CPS
        cidxv = jnp.full((_L,), _BIG, jnp.int32)
        for k in range(_CPS):
            cm = cmax_v[pl.ds((cbase + k) * _L, _L)]
            cidxv = jnp.minimum(cidxv, jnp.where(cm == big_m, cbase + k, _BIG))
        cidx = jnp.min(cidxv)

        base = cidx * _C
        idxacc = jnp.full((_L,), _BIG, jnp.int32)
        for i in range(_NVPC):
            v = row_v[pl.ds(base + i * _L, _L)]
            idxacc = jnp.minimum(
                idxacc, jnp.where(v == big_m, base + i * _L + iota, _BIG)
            )
        gidx = jnp.min(idxacc)
        picks.append(gidx)

        widx = gidx - _RADIUS + iota
        wmask = (iota < 2 * _RADIUS) & (gidx >= _RADIUS) & (widx < _N)
        widx_c = jnp.clip(widx, 0, _N - 1)
        plsc.store_scatter(row_v, [widx_c], m_v, mask=wmask)

        if t < _TOPK - 1:
            ws = jnp.maximum(gidx - _RADIUS, 0)
            we = jnp.minimum(gidx + _RADIUS, _N) - 1
            ca = ws // _C
            cb = we // _C
            _rescan_chunk(row_v, cmax_v, ca)
            _rescan_chunk(row_v, cmax_v, cb)
            _rebuild_super(cmax_v, smax_v, ca // _CPS)
            _rebuild_super(cmax_v, smax_v, cb // _CPS)

    return picks


def _pick_vec_mask(picks, iota):
    pv = jnp.full((_L,), picks[0], jnp.int32)
    vmask = iota == 0
    for t in range(1, _TOPK):
        pv = jnp.where(iota == t, picks[t], pv)
        dup = picks[t] == picks[0]
        for s in range(1, t):
            dup = dup | (picks[t] == picks[s])
        vmask = vmask | ((iota == t) & jnp.logical_not(dup))
    return pv, vmask


@functools.partial(
    pl.kernel,
    out_type=jax.ShapeDtypeStruct((_ROWS, _N), jnp.float32),
    mesh=_mesh,
    compiler_params=pltpu.CompilerParams(needs_layout_passes=False),
    scratch_types=[
        pltpu.VMEM((_N,), jnp.float32),
        pltpu.VMEM((_N,), jnp.float32),
        pltpu.VMEM((_N,), jnp.float32),
        pltpu.VMEM((_NCHUNK * _L,), jnp.float32),
        pltpu.VMEM((_NSUP * _L,), jnp.float32),
        pltpu.SemaphoreType.DMA,
        pltpu.SemaphoreType.DMA,
        pltpu.SemaphoreType.DMA,
    ],
)
def _wta_sc(x_hbm, out_hbm, rowa_v, rowb_v, out_v, cmax_v, smax_v,
            sem_a, sem_b, sem_o):
    wid = lax.axis_index("s") * 2 + lax.axis_index("c")
    ra = wid * 2
    rb = ra + 1
    iota = lax.iota(jnp.int32, _L)
    zero_v = jnp.zeros((_L,), jnp.float32)
    one_v = jnp.ones((_L,), jnp.float32)

    dma_a = pltpu.async_copy(x_hbm.at[ra], rowa_v, sem_a)
    dma_b = pltpu.async_copy(x_hbm.at[rb], rowb_v, sem_b)

    def _zbody(i, carry):
        for u in range(16):
            out_v[pl.ds(i * 256 + u * _L, _L)] = zero_v
        return carry

    lax.fori_loop(0, _N // 256, _zbody, jnp.int32(0))

    picks_a = _compute_picks(rowa_v, cmax_v, smax_v, iota, dma_a)
    pva, maska = _pick_vec_mask(picks_a, iota)
    plsc.store_scatter(out_v, [pva], one_v, mask=maska)
    out_a = pltpu.async_copy(out_v, out_hbm.at[ra], sem_o)

    picks_b = _compute_picks(rowb_v, cmax_v, smax_v, iota, dma_b)
    out_a.wait()
    plsc.store_scatter(out_v, [pva], zero_v, mask=maska)
    pvb, maskb = _pick_vec_mask(picks_b, iota)
    plsc.store_scatter(out_v, [pvb], one_v, mask=maskb)
    pltpu.sync_copy(out_v, out_hbm.at[rb])


def kernel(x):
    return _wta_sc(x)
